# baseline (speedup 1.0000x reference)
import functools

import jax
import jax.numpy as jnp
from jax.experimental import pallas as pl
from jax.experimental.pallas import tpu as pltpu

_MIB = 1024 * 1024
_TKM = 384


def _round_up(a: int, b: int) -> int:
    return (a + b - 1) // b * b


def _glu_mlp_kernel(x_ref, wgv_hbm, wo_hbm, o_ref,
                    wgv_buf, wo_buf, wgv_sem, wo_sem,
                    *, tkm, n_m, n_tiles):
    i = pl.program_id(0)
    H = o_ref.shape[1]

    def start_chunk(m, slot):
        pltpu.make_async_copy(
            wgv_hbm.at[:, pl.ds(m * 2 * tkm, 2 * tkm)],
            wgv_buf.at[slot], wgv_sem.at[slot]).start()
        pltpu.make_async_copy(
            wo_hbm.at[pl.ds(m * tkm, tkm), :],
            wo_buf.at[slot], wo_sem.at[slot]).start()

    base = i * n_m

    @pl.when(i == 0)
    def _prologue():
        start_chunk(0, 0)
        start_chunk(1, 1)

    def body(m, _):
        slot = jax.lax.rem(base + m, 2)
        pltpu.make_async_copy(wgv_buf.at[slot], wgv_buf.at[slot],
                              wgv_sem.at[slot]).wait()
        pltpu.make_async_copy(wo_buf.at[slot], wo_buf.at[slot],
                              wo_sem.at[slot]).wait()

        fused = jnp.dot(x_ref[...], wgv_buf[slot],
                        preferred_element_type=jnp.float32)
        gate = fused[:, :tkm]
        value = fused[:, tkm:]
        h = gate * jax.nn.sigmoid(gate) * value

        n_chunks = 4
        cw = H // n_chunks
        for c in range(n_chunks):
            sl = slice(c * cw, (c + 1) * cw)
            part = jnp.dot(h, wo_buf[slot][:, sl],
                           preferred_element_type=jnp.float32)
            prev = jnp.where(m == 0, 0.0, o_ref[:, sl])
            o_ref[:, sl] = part + prev

        nxt = base + m + 2

        @pl.when(nxt < n_tiles * n_m)
        def _prefetch():
            start_chunk(jax.lax.rem(m + 2, n_m), slot)

        return 0

    jax.lax.fori_loop(0, n_m, body, 0)


@jax.jit
def kernel(x, wgv, wo):
    H = x.shape[-1]
    lead_shape = x.shape[:-1]
    m_pad = wo.shape[0]
    tkm = _TKM
    n_m = m_pad // tkm

    x2d = x.reshape(-1, H)
    N = x2d.shape[0]

    tm = min(1024, max(128, _round_up(N, 128)))
    n_pad = _round_up(N, tm)
    if n_pad != N:
        x2d = jnp.pad(x2d, ((0, n_pad - N), (0, 0)))
    n_tiles = n_pad // tm

    cost = pl.CostEstimate(
        flops=6 * N * H * m_pad,
        transcendentals=N * m_pad,
        bytes_accessed=(2 * N * H * 4) + 3 * H * m_pad * 4 * n_tiles,
    )

    out2d = pl.pallas_call(
        functools.partial(_glu_mlp_kernel, tkm=tkm, n_m=n_m, n_tiles=n_tiles),
        out_shape=jax.ShapeDtypeStruct((n_pad, H), jnp.float32),
        grid_spec=pltpu.PrefetchScalarGridSpec(
            num_scalar_prefetch=0,
            grid=(n_tiles,),
            in_specs=[
                pl.BlockSpec((tm, H), lambda i: (i, 0)),
                pl.BlockSpec(memory_space=pl.ANY),
                pl.BlockSpec(memory_space=pl.ANY),
            ],
            out_specs=pl.BlockSpec((tm, H), lambda i: (i, 0)),
            scratch_shapes=[
                pltpu.VMEM((2, H, 2 * tkm), jnp.float32),
                pltpu.VMEM((2, tkm, H), jnp.float32),
                pltpu.SemaphoreType.DMA((2,)),
                pltpu.SemaphoreType.DMA((2,)),
            ],
        ),
        compiler_params=pltpu.CompilerParams(
            dimension_semantics=("arbitrary",),
            vmem_limit_bytes=60 * _MIB,
        ),
        cost_estimate=cost,
    )(x2d, wgv, wo)

    if n_pad != N:
        out2d = out2d[:N]
    return out2d.reshape(*lead_shape, H)


# speedup vs baseline: 1.1268x; 1.1268x over previous
import functools

import jax
import jax.numpy as jnp
from jax.experimental import pallas as pl
from jax.experimental.pallas import tpu as pltpu

_MIB = 1024 * 1024
_TKM = 384


def _round_up(a: int, b: int) -> int:
    return (a + b - 1) // b * b


def _glu_mlp_kernel(x_ref, wgv_hbm, wo_hbm, o_ref,
                    wgv_buf, wo_buf, wgv_sem, wo_sem,
                    *, tkm, n_m, n_tiles):
    i = pl.program_id(0)
    H = o_ref.shape[1]

    def start_chunk(m, slot):
        pltpu.make_async_copy(
            wgv_hbm.at[:, pl.ds(m * 2 * tkm, 2 * tkm)],
            wgv_buf.at[slot], wgv_sem.at[slot]).start()
        pltpu.make_async_copy(
            wo_hbm.at[pl.ds(m * tkm, tkm), :],
            wo_buf.at[slot], wo_sem.at[slot]).start()

    base = i * n_m

    @pl.when(i == 0)
    def _prologue():
        start_chunk(0, 0)
        start_chunk(1, 1)

    def body(m, _):
        slot = jax.lax.rem(base + m, 2)
        pltpu.make_async_copy(wgv_buf.at[slot], wgv_buf.at[slot],
                              wgv_sem.at[slot]).wait()
        pltpu.make_async_copy(wo_buf.at[slot], wo_buf.at[slot],
                              wo_sem.at[slot]).wait()

        n_chunks = 2
        cw = H // n_chunks
        n_r = 4
        tr = x_ref.shape[0] // n_r
        for r in range(n_r):
            rs = slice(r * tr, (r + 1) * tr)
            fused = jnp.dot(x_ref[rs, :], wgv_buf[slot],
                            preferred_element_type=jnp.float32)
            gate = fused[:, :tkm]
            value = fused[:, tkm:]
            h = gate * jax.nn.sigmoid(gate) * value
            for c in range(n_chunks):
                sl = slice(c * cw, (c + 1) * cw)
                part = jnp.dot(h, wo_buf[slot][:, sl],
                               preferred_element_type=jnp.float32)
                prev = jnp.where(m == 0, 0.0, o_ref[rs, sl])
                o_ref[rs, sl] = part + prev

        nxt = base + m + 2

        @pl.when(nxt < n_tiles * n_m)
        def _prefetch():
            start_chunk(jax.lax.rem(m + 2, n_m), slot)

        return 0

    jax.lax.fori_loop(0, n_m, body, 0)


@jax.jit
def kernel(x, wgv, wo):
    H = x.shape[-1]
    lead_shape = x.shape[:-1]
    m_pad = wo.shape[0]
    tkm = _TKM
    n_m = m_pad // tkm

    x2d = x.reshape(-1, H)
    N = x2d.shape[0]

    tm = min(1024, max(128, _round_up(N, 128)))
    n_pad = _round_up(N, tm)
    if n_pad != N:
        x2d = jnp.pad(x2d, ((0, n_pad - N), (0, 0)))
    n_tiles = n_pad // tm

    cost = pl.CostEstimate(
        flops=6 * N * H * m_pad,
        transcendentals=N * m_pad,
        bytes_accessed=(2 * N * H * 4) + 3 * H * m_pad * 4 * n_tiles,
    )

    out2d = pl.pallas_call(
        functools.partial(_glu_mlp_kernel, tkm=tkm, n_m=n_m, n_tiles=n_tiles),
        out_shape=jax.ShapeDtypeStruct((n_pad, H), jnp.float32),
        grid_spec=pltpu.PrefetchScalarGridSpec(
            num_scalar_prefetch=0,
            grid=(n_tiles,),
            in_specs=[
                pl.BlockSpec((tm, H), lambda i: (i, 0)),
                pl.BlockSpec(memory_space=pl.ANY),
                pl.BlockSpec(memory_space=pl.ANY),
            ],
            out_specs=pl.BlockSpec((tm, H), lambda i: (i, 0)),
            scratch_shapes=[
                pltpu.VMEM((2, H, 2 * tkm), jnp.float32),
                pltpu.VMEM((2, tkm, H), jnp.float32),
                pltpu.SemaphoreType.DMA((2,)),
                pltpu.SemaphoreType.DMA((2,)),
            ],
        ),
        compiler_params=pltpu.CompilerParams(
            dimension_semantics=("arbitrary",),
            vmem_limit_bytes=60 * _MIB,
        ),
        cost_estimate=cost,
    )(x2d, wgv, wo)

    if n_pad != N:
        out2d = out2d[:N]
    return out2d.reshape(*lead_shape, H)
